# baseline (speedup 1.0000x reference)
import functools

import jax
import jax.numpy as jnp
from jax import lax
from jax.experimental import pallas as pl
from jax.experimental.pallas import tpu as pltpu
from jax.experimental.pallas import tpu_sc as plsc

_K = 256
_B = 64
_N = 8192
_L = 16
_NV = _N // _L
_NC = 2
_NS = 16
_ROWS_PER_TILE = _B // (_NC * _NS)


def _mono_key(bits):
    return bits ^ (jnp.right_shift(bits, 31) & jnp.int32(0x7FFFFFFF))


def _sc_select_side(key_ref, hist_ref, tot_ref, k_start):
    iota = lax.iota(jnp.int32, _L)
    ones = jnp.ones((_L,), jnp.int32)

    def zero_hist():
        z = jnp.zeros((_L,), jnp.int32)

        def zbody(i, carry):
            hist_ref[pl.ds(i * _L, _L)] = z
            return carry

        lax.fori_loop(0, _L * 256 // _L, zbody, jnp.int32(0))

    def combine_and_pick(k_rem):
        def cbody(j, carry):
            acc = hist_ref[pl.ds(j * _L, _L)]
            for l in range(1, _L):
                acc = acc + hist_ref[pl.ds(l * 256 + j * _L, _L)]
            tot_ref[pl.ds(j * _L, _L)] = acc
            return carry

        lax.fori_loop(0, _L, cbody, jnp.int32(0))

        def sbody(i, carry):
            found, b_star, running = carry
            g = _L - 1 - i
            v = tot_ref[pl.ds(g * _L, _L)]
            rv = jnp.flip(v)
            c = plsc.cumsum(rv) + running
            crossed = c >= k_rem
            anyc = jnp.any(crossed)
            f = jnp.max(plsc.all_reduce_ffs(crossed))
            hit = jnp.logical_and(anyc, found == 0)
            b_star = jnp.where(hit, g * _L + (_L - 1) - f, b_star)
            found = jnp.where(hit, jnp.int32(1), found)
            return found, b_star, jnp.max(c)

        _, b_star, _ = lax.fori_loop(
            0, _L, sbody, (jnp.int32(0), jnp.int32(0), jnp.int32(0)))

        def abody(g, above):
            v = tot_ref[pl.ds(g * _L, _L)]
            m = (g * _L + iota) > b_star
            return above + jnp.sum(jnp.where(m, v, 0))

        above = lax.fori_loop(0, _L, abody, jnp.int32(0))
        return b_star, k_rem - above

    b1, k_rem = combine_and_pick(k_start)

    prefix = b1 - 128
    shifts = (16, 8, 0)
    for lvl in range(3):
        sh = shifts[lvl]
        zero_hist()

        def hbody(i, carry, _sh=sh, _prefix=prefix):
            v = key_ref[pl.ds(i * _L, _L)]
            sel = jnp.right_shift(v, _sh + 8) == _prefix
            b = jnp.right_shift(v, _sh) & jnp.int32(0xFF)
            plsc.addupdate_scatter(hist_ref, [iota * 256 + b], ones, mask=sel)
            return carry

        lax.fori_loop(0, _NV, hbody, jnp.int32(0))
        b_lvl, k_rem = combine_and_pick(k_rem)
        prefix = jnp.left_shift(prefix, 8) | b_lvl

    t = prefix
    k_eq = k_rem

    def tie_body(i, carry):
        running, j_star = carry
        v = key_ref[pl.ds(i * _L, _L)]
        m = v == t
        mi = m.astype(jnp.int32)
        cm = plsc.cumsum(mi)
        sel = jnp.logical_and(m, cm == (k_eq - running))
        anyc = jnp.any(sel)
        lane = jnp.max(plsc.all_reduce_ffs(sel))
        j_new = jnp.where(jnp.logical_and(anyc, j_star < 0),
                          i * _L + lane + 1, j_star)
        return running + jnp.max(cm), j_new

    _, j_star = lax.fori_loop(0, _NV, tie_body,
                              (jnp.int32(0), jnp.int32(-1)))
    return t, j_star


def _sc_body(scores_hbm, thr_hbm, x_v, keyl_v, keys_v, histl_ref, hists_ref,
             tot_ref, out_v, sem):
    wid = lax.axis_index("s") * _NC + lax.axis_index("c")
    iota = lax.iota(jnp.int32, _L)
    ones = jnp.ones((_L,), jnp.int32)

    for r in range(_ROWS_PER_TILE):
        row = wid * _ROWS_PER_TILE + r
        pltpu.async_copy(scores_hbm.at[row], x_v, sem).wait()

        def p0(i, carry):
            x = x_v[pl.ds(i * _L, _L)]
            bl = lax.bitcast_convert_type(x, jnp.int32)
            kl = _mono_key(bl)
            keyl_v[pl.ds(i * _L, _L)] = kl
            loser = jnp.sign(x) * (1.0 - x)
            ks = _mono_key(lax.bitcast_convert_type(loser, jnp.int32))
            keys_v[pl.ds(i * _L, _L)] = ks
            b_l = jnp.right_shift(kl, 24) + 128
            b_s = jnp.right_shift(ks, 24) + 128
            plsc.addupdate_scatter(histl_ref, [iota * 256 + b_l], ones)
            plsc.addupdate_scatter(hists_ref, [iota * 256 + b_s], ones)
            return carry

        z = jnp.zeros((_L,), jnp.int32)

        def zbody(i, carry):
            histl_ref[pl.ds(i * _L, _L)] = z
            hists_ref[pl.ds(i * _L, _L)] = z
            return carry

        lax.fori_loop(0, _L * 256 // _L, zbody, jnp.int32(0))
        lax.fori_loop(0, _NV, p0, jnp.int32(0))

        t_l, j_l = _sc_select_side(keyl_v, histl_ref, tot_ref, jnp.int32(_K))
        t_s, j_s = _sc_select_side(keys_v, hists_ref, tot_ref, jnp.int32(_K))

        vec = jnp.where(iota == 0, t_l,
              jnp.where(iota == 1, j_l,
              jnp.where(iota == 2, t_s,
              jnp.where(iota == 3, j_s, jnp.int32(0)))))
        out_v[...] = vec
        pltpu.async_copy(out_v, thr_hbm.at[row], sem).wait()


_SC_MESH = plsc.VectorSubcoreMesh(core_axis_name="c", subcore_axis_name="s")


@functools.partial(
    pl.kernel,
    out_type=jax.ShapeDtypeStruct((_B, _L), jnp.int32),
    mesh=_SC_MESH,
    compiler_params=pltpu.CompilerParams(needs_layout_passes=False),
    scratch_types=[
        pltpu.VMEM((_N,), jnp.float32),
        pltpu.VMEM((_N,), jnp.int32),
        pltpu.VMEM((_N,), jnp.int32),
        pltpu.VMEM((_L * 256,), jnp.int32),
        pltpu.VMEM((_L * 256,), jnp.int32),
        pltpu.VMEM((256,), jnp.int32),
        pltpu.VMEM((_L,), jnp.int32),
        pltpu.SemaphoreType.DMA,
    ],
)
def _sc_thresholds(scores_hbm, thr_hbm, x_v, keyl_v, keys_v, histl_ref,
                   hists_ref, tot_ref, out_v, sem):
    _sc_body(scores_hbm, thr_hbm, x_v, keyl_v, keys_v, histl_ref, hists_ref,
             tot_ref, out_v, sem)


def _tc_body(scores_ref, thr_ref, w_ref, probs_ref):
    x = scores_ref[...]
    B, N = x.shape
    thr = thr_ref[...]
    idx = lax.broadcasted_iota(jnp.int32, (B, N), 1)

    def side_mask(key, t, j):
        return (key > t) | ((key == t) & (idx < j))

    key_l = _mono_key(lax.bitcast_convert_type(x, jnp.int32))
    mask_l = side_mask(key_l, thr[:, 0:1], thr[:, 1:2])

    loser = jnp.sign(x) * (1.0 - x)
    key_s = _mono_key(lax.bitcast_convert_type(loser, jnp.int32))
    mask_s = side_mask(key_s, thr[:, 2:3], thr[:, 3:4])

    rowmax = jnp.max(x, axis=1, keepdims=True)
    e = jnp.exp(x - rowmax)
    probs_ref[...] = e / jnp.sum(e, axis=1, keepdims=True)

    denom_l = jnp.sum(jnp.where(mask_l, e, 0.0), axis=1, keepdims=True)
    w_ref[:, :N] = jnp.where(mask_l, e / denom_l, 0.0)

    lmax = jnp.max(loser, axis=1, keepdims=True)
    el = jnp.exp(loser - lmax)
    denom_s = jnp.sum(jnp.where(mask_s, el, 0.0), axis=1, keepdims=True)
    w_ref[:, N:] = jnp.where(mask_s, el / denom_s, 0.0)


@jax.jit
def _run(scores):
    B, N = scores.shape
    thr = _sc_thresholds(scores)
    w, probs = pl.pallas_call(
        _tc_body,
        out_shape=(
            jax.ShapeDtypeStruct((B, 2 * N), scores.dtype),
            jax.ShapeDtypeStruct((B, N), scores.dtype),
        ),
    )(scores, thr)
    return w, probs


def kernel(scores, G):
    B, _ = scores.shape
    w, probs = _run(scores)
    rho = jnp.full((B,), 0.5, dtype=scores.dtype)
    return (w, rho, probs)


# speedup vs baseline: 5.1278x; 5.1278x over previous
import functools

import jax
import jax.numpy as jnp
from jax.experimental import pallas as pl
from jax.experimental.pallas import tpu as pltpu

_K = 256


def _monotonic_key(v):
    bits = jax.lax.bitcast_convert_type(v, jnp.int32)
    return jnp.where(bits < 0, bits ^ jnp.int32(0x7FFFFFFF), bits)


def _count_ge(key, mid):
    return jnp.sum((key >= mid).astype(jnp.int32), axis=1, keepdims=True)


def _topk_masks(key_a, key_b, k):
    B, N = key_a.shape

    lo0 = jnp.full((B, 1), jnp.iinfo(jnp.int32).min, jnp.int32)
    hi0 = jnp.full((B, 1), jnp.iinfo(jnp.int32).max, jnp.int32)

    def val_step(_, carry):
        lo_a, hi_a, lo_b, hi_b = carry
        mid_a = (lo_a >> 1) + (hi_a >> 1) + (lo_a & hi_a & 1)
        mid_b = (lo_b >> 1) + (hi_b >> 1) + (lo_b & hi_b & 1)
        ge_a = _count_ge(key_a, mid_a) >= k
        ge_b = _count_ge(key_b, mid_b) >= k
        return (jnp.where(ge_a, mid_a, lo_a), jnp.where(ge_a, hi_a, mid_a),
                jnp.where(ge_b, mid_b, lo_b), jnp.where(ge_b, hi_b, mid_b))

    t_a, _, t_b, _ = jax.lax.fori_loop(0, 32, val_step, (lo0, hi0, lo0, hi0))

    gt_a = key_a > t_a
    gt_b = key_b > t_b
    eq_a = key_a == t_a
    eq_b = key_b == t_b
    keq_a = k - jnp.sum(gt_a.astype(jnp.int32), axis=1, keepdims=True)
    keq_b = k - jnp.sum(gt_b.astype(jnp.int32), axis=1, keepdims=True)
    ceq_a = jnp.sum(eq_a.astype(jnp.int32), axis=1, keepdims=True)
    ceq_b = jnp.sum(eq_b.astype(jnp.int32), axis=1, keepdims=True)
    idx = jax.lax.broadcasted_iota(jnp.int32, (B, N), 1)

    ties = jnp.any(ceq_a > keq_a) | jnp.any(ceq_b > keq_b)

    jlo0 = jnp.zeros((B, 1), jnp.int32)
    jhi0 = jnp.full((B, 1), N, jnp.int32)

    def idx_search(_):
        def idx_step(_, carry):
            jlo_a, jhi_a, jlo_b, jhi_b = carry
            jmid_a = (jlo_a + jhi_a) >> 1
            jmid_b = (jlo_b + jhi_b) >> 1
            c_a = jnp.sum((eq_a & (idx < jmid_a)).astype(jnp.int32),
                          axis=1, keepdims=True)
            c_b = jnp.sum((eq_b & (idx < jmid_b)).astype(jnp.int32),
                          axis=1, keepdims=True)
            ge_a = c_a >= keq_a
            ge_b = c_b >= keq_b
            return (jnp.where(ge_a, jlo_a, jmid_a),
                    jnp.where(ge_a, jmid_a, jhi_a),
                    jnp.where(ge_b, jlo_b, jmid_b),
                    jnp.where(ge_b, jmid_b, jhi_b))

        nbits = max(1, (N - 1).bit_length())
        _, jhi_a, _, jhi_b = jax.lax.fori_loop(0, nbits, idx_step,
                                               (jlo0, jhi0, jlo0, jhi0))
        return jhi_a, jhi_b

    jhi_a, jhi_b = jax.lax.cond(ties, idx_search,
                                lambda _: (jhi0, jhi0), None)
    mask_a = gt_a | (eq_a & (idx < jhi_a))
    mask_b = gt_b | (eq_b & (idx < jhi_b))
    return mask_a, mask_b


def _body(scores_ref, w_ref, probs_ref):
    x = scores_ref[...]
    B, N = x.shape

    loser = jnp.sign(x) * (1.0 - x)
    mask_l, mask_s = _topk_masks(_monotonic_key(x), _monotonic_key(loser), _K)

    rowmax = jnp.max(x, axis=1, keepdims=True)
    e = jnp.exp(x - rowmax)
    probs_ref[...] = e / jnp.sum(e, axis=1, keepdims=True)

    denom_l = jnp.sum(jnp.where(mask_l, e, 0.0), axis=1, keepdims=True)
    w_ref[:, :N] = jnp.where(mask_l, e / denom_l, 0.0)

    lmax = jnp.max(loser, axis=1, keepdims=True)
    el = jnp.exp(loser - lmax)
    denom_s = jnp.sum(jnp.where(mask_s, el, 0.0), axis=1, keepdims=True)
    w_ref[:, N:] = jnp.where(mask_s, el / denom_s, 0.0)


@functools.partial(jax.jit, static_argnames=("interpret",))
def _run(scores, interpret=False):
    B, N = scores.shape
    w, probs = pl.pallas_call(
        _body,
        out_shape=(
            jax.ShapeDtypeStruct((B, 2 * N), scores.dtype),
            jax.ShapeDtypeStruct((B, N), scores.dtype),
        ),
        interpret=interpret,
    )(scores)
    return w, probs


def kernel(scores, G):
    B, _ = scores.shape
    w, probs = _run(scores)
    rho = jnp.full((B,), 0.5, dtype=scores.dtype)
    return (w, rho, probs)
